# speedup vs baseline: 15.7173x; 15.7173x over previous
import jax
import jax.numpy as jnp
from jax.experimental import pallas as pl
from jax.experimental.pallas import tpu as pltpu

_E = 256
_C = 128
_S = 64
_M = _E // _S
_ROW = _M * _C
_NREP = 64
_BBLK = 64


def _body(sel_ref, lat_ref, embt_ref, out_ref):
    sel = sel_ref[...]
    jcol = jax.lax.broadcasted_iota(jnp.int32, (_BBLK, _NREP), 1)
    for s in range(_S):
        onehot = (sel[:, s][:, None] == jcol).astype(jnp.bfloat16)
        g = jax.lax.dot_general(
            onehot, embt_ref[s],
            (((1,), (0,)), ((), ())),
            preferred_element_type=jnp.float32,
        )
        sl = slice(s * _ROW, (s + 1) * _ROW)
        out_ref[:, sl] = lat_ref[:, sl] + g


def kernel(latents, selections, emb):
    b = latents.shape[0]
    lat2d = latents.reshape(b, _E * _C)
    sel = selections.astype(jnp.int32)
    embt = (
        emb.reshape(_S, _M, _NREP, _C)
        .transpose(0, 2, 1, 3)
        .reshape(_S, _NREP, _ROW)
        .astype(jnp.bfloat16)
    )
    out = pl.pallas_call(
        _body,
        grid=(b // _BBLK,),
        in_specs=[
            pl.BlockSpec((_BBLK, _S), lambda i: (i, 0)),
            pl.BlockSpec((_BBLK, _E * _C), lambda i: (i, 0)),
            pl.BlockSpec((_S, _NREP, _ROW), lambda i: (0, 0, 0)),
        ],
        out_specs=pl.BlockSpec((_BBLK, _E * _C), lambda i: (i, 0)),
        out_shape=jax.ShapeDtypeStruct((b, _E * _C), jnp.float32),
        compiler_params=pltpu.CompilerParams(
            dimension_semantics=("arbitrary",),
        ),
    )(sel, lat2d, embt)
    return out.reshape(latents.shape)


# speedup vs baseline: 39.1344x; 2.4899x over previous
import jax
import jax.numpy as jnp
from jax.experimental import pallas as pl
from jax.experimental.pallas import tpu as pltpu

_E = 256
_C = 128
_S = 64
_M = _E // _S
_ROW = _M * _C
_NREP = 64
_BBLK = 64


def _body(sel_ref, lat_ref, embt_ref, out_ref):
    sel = sel_ref[...]
    jcol = jax.lax.broadcasted_iota(jnp.int32, (_BBLK, _NREP), 1)
    for s in range(_S):
        onehot = (sel[:, s][:, None] == jcol).astype(jnp.bfloat16)
        g = jax.lax.dot_general(
            onehot, embt_ref[s],
            (((1,), (0,)), ((), ())),
            preferred_element_type=jnp.float32,
        )
        sl = slice(_M * s, _M * (s + 1))
        out_ref[:, sl, :] = lat_ref[:, sl, :] + g.reshape(_BBLK, _M, _C)


def kernel(latents, selections, emb):
    b = latents.shape[0]
    sel = selections.astype(jnp.int32)
    embt = (
        emb.reshape(_S, _M, _NREP, _C)
        .transpose(0, 2, 1, 3)
        .reshape(_S, _NREP, _ROW)
        .astype(jnp.bfloat16)
    )
    return pl.pallas_call(
        _body,
        grid=(b // _BBLK,),
        in_specs=[
            pl.BlockSpec((_BBLK, _S), lambda i: (i, 0)),
            pl.BlockSpec((_BBLK, _E, _C), lambda i: (i, 0, 0)),
            pl.BlockSpec((_S, _NREP, _ROW), lambda i: (0, 0, 0)),
        ],
        out_specs=pl.BlockSpec((_BBLK, _E, _C), lambda i: (i, 0, 0)),
        out_shape=jax.ShapeDtypeStruct((b, _E, _C), jnp.float32),
        compiler_params=pltpu.CompilerParams(
            dimension_semantics=("arbitrary",),
        ),
    )(sel, latents, embt)
